# Initial kernel scaffold; baseline (speedup 1.0000x reference)
import jax
import jax.numpy as jnp
from jax.experimental import pallas as pl


def kernel(seq1, seq2, adj, sparse, msk, samp_bias1, samp_bias2, W_fc, b_gcn, prelu_a, W_bil, b_bil):
    raise NotImplementedError("write your pallas kernel here")


# speedup vs baseline: 1.8596x; 1.8596x over previous
import jax
import jax.numpy as jnp
from jax.experimental import pallas as pl
from jax.experimental.pallas import tpu as pltpu

N = 10000
N_IN = 128
N_H = 64
BM = 400


def _dgi_kernel(seq1_ref, seq2_ref, adj_ref, wfc_ref, b2_ref, a_ref,
                msk_ref, mskblk_ref, sb_ref, wbilt_ref, bbil_ref,
                out_ref, fts_ref, h_ref, csum_ref):
    g = pl.program_id(0)
    num_blocks = pl.num_programs(0)

    @pl.when(g == 0)
    def _init():
        fts_ref[:, :N_H] = jnp.dot(seq1_ref[...], wfc_ref[...],
                                   preferred_element_type=jnp.float32)
        fts_ref[:, N_H:] = jnp.dot(seq2_ref[...], wfc_ref[...],
                                   preferred_element_type=jnp.float32)
        csum_ref[...] = jnp.zeros_like(csum_ref)

    out = jnp.dot(adj_ref[...], fts_ref[...],
                  preferred_element_type=jnp.float32)
    out = out + b2_ref[...]
    a = a_ref[0, 0]
    h = jnp.where(out > 0, out, a * out)
    h_ref[pl.ds(g * BM, BM), :] = h

    msk_blk = mskblk_ref[0]
    csum_ref[...] += jnp.dot(msk_blk, h,
                             preferred_element_type=jnp.float32)

    @pl.when(g == num_blocks - 1)
    def _finish():
        msk_total = jnp.sum(msk_ref[...])
        c = jax.nn.sigmoid(csum_ref[:, :N_H] / msk_total)
        v = jnp.dot(c, wbilt_ref[...],
                    preferred_element_type=jnp.float32)
        h1 = h_ref[:, :N_H]
        h2 = h_ref[:, N_H:]
        dn = (((1,), (1,)), ((), ()))
        sc1 = jax.lax.dot_general(v, h1, dn,
                                  preferred_element_type=jnp.float32)
        sc2 = jax.lax.dot_general(v, h2, dn,
                                  preferred_element_type=jnp.float32)
        b = bbil_ref[0, 0]
        out_ref[0:1, :] = sc1 + b + sb_ref[0:1, :]
        out_ref[1:2, :] = sc2 + b + sb_ref[1:2, :]


def kernel(seq1, seq2, adj, sparse, msk, samp_bias1, samp_bias2,
           W_fc, b_gcn, prelu_a, W_bil, b_bil):
    del sparse
    seq1_2d = seq1.reshape(N, N_IN)
    seq2_2d = seq2.reshape(N, N_IN)
    adj_2d = adj.reshape(N, N)
    wfc_t = W_fc.T
    b2 = jnp.concatenate([b_gcn, b_gcn]).reshape(1, 2 * N_H)
    a2 = prelu_a.reshape(1, 1)
    msk_2d = msk.reshape(1, N)
    msk_blocks = msk.reshape(N // BM, 1, BM)
    sb = jnp.concatenate([samp_bias1, samp_bias2], axis=0)
    wbil_t = W_bil.reshape(N_H, N_H).T
    bbil_2d = b_bil.reshape(1, 1)

    grid = (N // BM,)
    full = lambda g: (0, 0)
    out = pl.pallas_call(
        _dgi_kernel,
        grid=grid,
        in_specs=[
            pl.BlockSpec((N, N_IN), full),
            pl.BlockSpec((N, N_IN), full),
            pl.BlockSpec((BM, N), lambda g: (g, 0)),
            pl.BlockSpec((N_IN, N_H), full),
            pl.BlockSpec((1, 2 * N_H), full),
            pl.BlockSpec((1, 1), full),
            pl.BlockSpec((1, N), full),
            pl.BlockSpec((1, 1, BM), lambda g: (g, 0, 0)),
            pl.BlockSpec((2, N), full),
            pl.BlockSpec((N_H, N_H), full),
            pl.BlockSpec((1, 1), full),
        ],
        out_specs=pl.BlockSpec((2, N), full),
        out_shape=jax.ShapeDtypeStruct((2, N), jnp.float32),
        scratch_shapes=[
            pltpu.VMEM((N, 2 * N_H), jnp.float32),
            pltpu.VMEM((N, 2 * N_H), jnp.float32),
            pltpu.VMEM((1, 2 * N_H), jnp.float32),
        ],
        compiler_params=pltpu.CompilerParams(
            dimension_semantics=("arbitrary",),
            vmem_limit_bytes=110 * 1024 * 1024,
        ),
    )(seq1_2d, seq2_2d, adj_2d, wfc_t, b2, a2, msk_2d, msk_blocks, sb,
      wbil_t, bbil_2d)

    return out.reshape(1, 2 * N)
